# baseline (speedup 1.0000x reference)
import functools

import jax
import jax.numpy as jnp
from jax import lax
from jax.experimental import pallas as pl
from jax.experimental.pallas import tpu as pltpu
from jax.experimental.pallas import tpu_sc as plsc

N = 10000
E = 320000
D = 128
G = 64

NC = 2
NS = 16
DH = D // NC
EPW = E // NS
WIN = 125
NWIN = EPW // WIN
CHW = 40
NCH = NWIN // CHW
NBUF = 4
RPT = 624
TAIL = N - NS * RPT

RB = 1000
NRB = N // RB


def _sc_gather_segsum(xh, src_r, dst_r):
    mesh = plsc.VectorSubcoreMesh(
        core_axis_name="c", subcore_axis_name="s", num_cores=NC)

    @functools.partial(
        pl.kernel,
        out_type=jax.ShapeDtypeStruct((NC, N, DH), jnp.float32),
        mesh=mesh,
        compiler_params=pltpu.CompilerParams(use_tc_tiling_on_sc=False),
        scratch_types=(
            [pltpu.VMEM((CHW, WIN), jnp.int32),
             pltpu.VMEM((CHW, WIN), jnp.int32)]
            + [pltpu.VMEM((WIN, DH), jnp.float32)] * NBUF
            + [pltpu.VMEM_SHARED((N, DH), jnp.float32)]
            + [pltpu.SemaphoreType.DMA] * (2 * NBUF)
        ),
    )
    def k(x_hbm, src_hbm, dst_hbm, out_hbm,
          src_v, dst_v, *rest):
        rows = rest[:NBUF]
        agg_sh = rest[NBUF]
        sems = rest[NBUF + 1:NBUF + 1 + NBUF]
        ssems = rest[NBUF + 1 + NBUF:]
        cid = lax.axis_index("c")
        sid = lax.axis_index("s")
        xc = x_hbm.at[cid]

        zv = jnp.zeros((16,), jnp.float32)

        @pl.loop(0, WIN)
        def _(r):
            for c in range(DH // 16):
                rows[0][r, pl.ds(c * 16, 16)] = zv

        for t in range(RPT // 104):
            pltpu.sync_copy(rows[0].at[pl.ds(0, 104)],
                            agg_sh.at[pl.ds(sid * RPT + t * 104, 104)])

        @pl.when(sid == 0)
        def _():
            pltpu.sync_copy(rows[0].at[pl.ds(0, TAIL)],
                            agg_sh.at[pl.ds(NS * RPT, TAIL)])

        plsc.subcore_barrier()

        @pl.loop(0, NCH)
        def _(ci):
            pltpu.sync_copy(src_hbm.at[sid].at[pl.ds(ci * CHW, CHW)], src_v)
            pltpu.sync_copy(dst_hbm.at[sid].at[pl.ds(ci * CHW, CHW)], dst_v)
            for k in range(2):
                pltpu.make_async_copy(
                    xc.at[src_v.at[k]], rows[k], sems[k]).start()

            @pl.loop(0, CHW, step=NBUF)
            def _(j):
                for k in range(NBUF):
                    kn = (k + 2) % NBUF
                    pltpu.make_async_copy(
                        xc.at[src_v.at[j + k]], rows[k], sems[k]).wait()
                    pltpu.async_copy(rows[k], agg_sh.at[dst_v.at[j + k]],
                                     ssems[k], add=True)

                    @pl.when(j + k + 2 < CHW)
                    def _():
                        @pl.when(j + k - 2 >= 0)
                        def _():
                            pltpu.make_async_copy(
                                rows[kn], agg_sh.at[dst_v.at[j + k - 2]],
                                ssems[kn]).wait()

                        pltpu.make_async_copy(
                            xc.at[src_v.at[j + k + 2]],
                            rows[kn], sems[kn]).start()

            for k in range(NBUF):
                pltpu.make_async_copy(
                    rows[k], agg_sh.at[dst_v.at[CHW - NBUF + k]],
                    ssems[k]).wait()

        plsc.subcore_barrier()
        pltpu.sync_copy(agg_sh.at[pl.ds(sid * RPT, RPT)],
                        out_hbm.at[cid].at[pl.ds(sid * RPT, RPT)])

        @pl.when(sid == 0)
        def _():
            pltpu.sync_copy(agg_sh.at[pl.ds(NS * RPT, TAIL)],
                            out_hbm.at[cid].at[pl.ds(NS * RPT, TAIL)])

    return k(xh, src_r, dst_r)


def _dot(a, b):
    return lax.dot_general(a, b, (((1,), (0,)), ((), ())),
                           preferred_element_type=jnp.float32)


def _tc_mlp(agg, h, w1, b1, w2, b2):
    def body(agg_ref, h_ref, w1_ref, b1_ref, w2_ref, b2_ref, out_ref):
        a = jnp.concatenate(
            [agg_ref[0] + h_ref[0], agg_ref[1] + h_ref[1]], axis=1)
        z = jnp.maximum(_dot(a, w1_ref[...]) + b1_ref[...], 0.0)
        h2 = jnp.maximum(_dot(z, w2_ref[...]) + b2_ref[...], 0.0)
        out_ref[0] = h2[:, :DH]
        out_ref[1] = h2[:, DH:]

    full = lambda *_: (0, 0)
    return pl.pallas_call(
        body,
        grid=(NRB,),
        in_specs=[
            pl.BlockSpec((NC, RB, DH), lambda i: (0, i, 0)),
            pl.BlockSpec((NC, RB, DH), lambda i: (0, i, 0)),
            pl.BlockSpec((D, D), full),
            pl.BlockSpec((1, D), full),
            pl.BlockSpec((D, D), full),
            pl.BlockSpec((1, D), full),
        ],
        out_specs=pl.BlockSpec((NC, RB, DH), lambda i: (0, i, 0)),
        out_shape=jax.ShapeDtypeStruct((NC, N, DH), jnp.float32),
    )(agg, h, w1, b1, w2, b2)


def _tc_pool(h, batch_r):
    def body(h_ref, batch_ref, out_ref, acc_ref, cnt_ref):
        i = pl.program_id(0)

        @pl.when(i == 0)
        def _():
            acc_ref[...] = jnp.zeros_like(acc_ref)
            cnt_ref[...] = jnp.zeros_like(cnt_ref)

        hh = jnp.concatenate([h_ref[0], h_ref[1]], axis=1)
        bb = batch_ref[0, 0, :]
        gids = lax.broadcasted_iota(jnp.int32, (RB, G), 1)
        onehot = (bb[:, None] == gids).astype(jnp.float32)
        acc_ref[...] += lax.dot_general(
            onehot, hh, (((0,), (0,)), ((), ())),
            preferred_element_type=jnp.float32)
        cnt_ref[...] += jnp.broadcast_to(
            jnp.sum(onehot, axis=0)[:, None], (G, D))

        @pl.when(i == NRB - 1)
        def _():
            out_ref[...] = acc_ref[...] / jnp.maximum(cnt_ref[...], 1.0)

    full = lambda *_: (0, 0)
    return pl.pallas_call(
        body,
        grid=(NRB,),
        in_specs=[
            pl.BlockSpec((NC, RB, DH), lambda i: (0, i, 0)),
            pl.BlockSpec((1, 1, RB), lambda i: (i, 0, 0)),
        ],
        out_specs=pl.BlockSpec((G, D), full),
        out_shape=jax.ShapeDtypeStruct((G, D), jnp.float32),
        scratch_shapes=[
            pltpu.VMEM((G, D), jnp.float32),
            pltpu.VMEM((G, D), jnp.float32),
        ],
    )(h, batch_r)


def kernel(x, edge_index, batch, W1a, b1a, W2a, b2a, W1b, b1b, W2b, b2b):
    src_r = edge_index[0].reshape(NS, NWIN, WIN)
    dst_r = edge_index[1].reshape(NS, NWIN, WIN)
    batch_r = batch.reshape(NRB, 1, RB)
    xh = jnp.stack([x[:, :DH], x[:, DH:]])

    w1s = jnp.stack([W1a, W1b])
    b1s = jnp.stack([b1a.reshape(1, D), b1b.reshape(1, D)])
    w2s = jnp.stack([W2a, W2b])
    b2s = jnp.stack([b2a.reshape(1, D), b2b.reshape(1, D)])

    def step(h, ws):
        w1, b1, w2, b2 = ws
        agg = _sc_gather_segsum(h, src_r, dst_r)
        return _tc_mlp(agg, h, w1, b1, w2, b2), None

    h2, _ = lax.scan(step, xh, (w1s, b1s, w2s, b2s))
    return _tc_pool(h2, batch_r)


# speedup vs baseline: 1.1291x; 1.1291x over previous
import functools

import jax
import jax.numpy as jnp
from jax import lax
from jax.experimental import pallas as pl
from jax.experimental.pallas import tpu as pltpu
from jax.experimental.pallas import tpu_sc as plsc

N = 10000
E = 320000
D = 128
G = 64

NC = 2
NS = 16
DH = D // NC
EPW = E // NS
WIN = 125
NWIN = EPW // WIN
CHW = 40
NCH = NWIN // CHW
NBUF = 4
RPT = 624
TAIL = N - NS * RPT

RB = 1000
NRB = N // RB


def _sc_gather_segsum(xh, src_r, dst_r):
    mesh = plsc.VectorSubcoreMesh(
        core_axis_name="c", subcore_axis_name="s", num_cores=NC)

    @functools.partial(
        pl.kernel,
        out_type=jax.ShapeDtypeStruct((NC, N, DH), jnp.float32),
        mesh=mesh,
        compiler_params=pltpu.CompilerParams(use_tc_tiling_on_sc=False),
        scratch_types=(
            [pltpu.VMEM((CHW, WIN), jnp.int32),
             pltpu.VMEM((CHW, WIN), jnp.int32)]
            + [pltpu.VMEM((WIN, DH), jnp.float32)] * NBUF
            + [pltpu.VMEM_SHARED((N, DH), jnp.float32)]
            + [pltpu.SemaphoreType.DMA] * NBUF
        ),
    )
    def k(x_hbm, src_hbm, dst_hbm, out_hbm,
          src_v, dst_v, *rest):
        rows = rest[:NBUF]
        agg_sh = rest[NBUF]
        sems = rest[NBUF + 1:]
        cid = lax.axis_index("c")
        sid = lax.axis_index("s")
        xc = x_hbm.at[cid]

        zv = jnp.zeros((16,), jnp.float32)

        @pl.loop(0, WIN)
        def _(r):
            for c in range(DH // 16):
                rows[0][r, pl.ds(c * 16, 16)] = zv

        for t in range(RPT // 104):
            pltpu.sync_copy(rows[0].at[pl.ds(0, 104)],
                            agg_sh.at[pl.ds(sid * RPT + t * 104, 104)])

        @pl.when(sid == 0)
        def _():
            pltpu.sync_copy(rows[0].at[pl.ds(0, TAIL)],
                            agg_sh.at[pl.ds(NS * RPT, TAIL)])

        plsc.subcore_barrier()

        @pl.loop(0, NCH)
        def _(ci):
            pltpu.sync_copy(src_hbm.at[sid].at[pl.ds(ci * CHW, CHW)], src_v)
            pltpu.sync_copy(dst_hbm.at[sid].at[pl.ds(ci * CHW, CHW)], dst_v)
            for k in range(NBUF):
                pltpu.make_async_copy(
                    xc.at[src_v.at[k]], rows[k], sems[k]).start()

            @pl.loop(0, CHW, step=NBUF)
            def _(j):
                for k in range(NBUF):
                    pltpu.make_async_copy(
                        xc.at[src_v.at[j + k]], rows[k], sems[k]).wait()
                    pltpu.sync_copy(rows[k], agg_sh.at[dst_v.at[j + k]],
                                    add=True)

                    @pl.when(j + k + NBUF < CHW)
                    def _():
                        pltpu.make_async_copy(
                            xc.at[src_v.at[j + k + NBUF]],
                            rows[k], sems[k]).start()

        plsc.subcore_barrier()
        pltpu.sync_copy(agg_sh.at[pl.ds(sid * RPT, RPT)],
                        out_hbm.at[cid].at[pl.ds(sid * RPT, RPT)])

        @pl.when(sid == 0)
        def _():
            pltpu.sync_copy(agg_sh.at[pl.ds(NS * RPT, TAIL)],
                            out_hbm.at[cid].at[pl.ds(NS * RPT, TAIL)])

    return k(xh, src_r, dst_r)


def _dot(a, b):
    return lax.dot_general(a, b, (((1,), (0,)), ((), ())),
                           preferred_element_type=jnp.float32)


def _tc_mlp(agg, h, w1, b1, w2, b2):
    def body(agg_ref, h_ref, w1_ref, b1_ref, w2_ref, b2_ref, out_ref):
        a = jnp.concatenate(
            [agg_ref[0] + h_ref[0], agg_ref[1] + h_ref[1]], axis=1)
        z = jnp.maximum(_dot(a, w1_ref[...]) + b1_ref[...], 0.0)
        h2 = jnp.maximum(_dot(z, w2_ref[...]) + b2_ref[...], 0.0)
        out_ref[0] = h2[:, :DH]
        out_ref[1] = h2[:, DH:]

    full = lambda *_: (0, 0)
    return pl.pallas_call(
        body,
        grid=(NRB,),
        in_specs=[
            pl.BlockSpec((NC, RB, DH), lambda i: (0, i, 0)),
            pl.BlockSpec((NC, RB, DH), lambda i: (0, i, 0)),
            pl.BlockSpec((D, D), full),
            pl.BlockSpec((1, D), full),
            pl.BlockSpec((D, D), full),
            pl.BlockSpec((1, D), full),
        ],
        out_specs=pl.BlockSpec((NC, RB, DH), lambda i: (0, i, 0)),
        out_shape=jax.ShapeDtypeStruct((NC, N, DH), jnp.float32),
    )(agg, h, w1, b1, w2, b2)


def _tc_pool(h, batch_r):
    def body(h_ref, batch_ref, out_ref, acc_ref, cnt_ref):
        i = pl.program_id(0)

        @pl.when(i == 0)
        def _():
            acc_ref[...] = jnp.zeros_like(acc_ref)
            cnt_ref[...] = jnp.zeros_like(cnt_ref)

        hh = jnp.concatenate([h_ref[0], h_ref[1]], axis=1)
        bb = batch_ref[0, 0, :]
        gids = lax.broadcasted_iota(jnp.int32, (RB, G), 1)
        onehot = (bb[:, None] == gids).astype(jnp.float32)
        acc_ref[...] += lax.dot_general(
            onehot, hh, (((0,), (0,)), ((), ())),
            preferred_element_type=jnp.float32)
        cnt_ref[...] += jnp.broadcast_to(
            jnp.sum(onehot, axis=0)[:, None], (G, D))

        @pl.when(i == NRB - 1)
        def _():
            out_ref[...] = acc_ref[...] / jnp.maximum(cnt_ref[...], 1.0)

    full = lambda *_: (0, 0)
    return pl.pallas_call(
        body,
        grid=(NRB,),
        in_specs=[
            pl.BlockSpec((NC, RB, DH), lambda i: (0, i, 0)),
            pl.BlockSpec((1, 1, RB), lambda i: (i, 0, 0)),
        ],
        out_specs=pl.BlockSpec((G, D), full),
        out_shape=jax.ShapeDtypeStruct((G, D), jnp.float32),
        scratch_shapes=[
            pltpu.VMEM((G, D), jnp.float32),
            pltpu.VMEM((G, D), jnp.float32),
        ],
    )(h, batch_r)


def kernel(x, edge_index, batch, W1a, b1a, W2a, b2a, W1b, b1b, W2b, b2b):
    src_r = edge_index[0].reshape(NS, NWIN, WIN)
    dst_r = edge_index[1].reshape(NS, NWIN, WIN)
    batch_r = batch.reshape(NRB, 1, RB)
    xh = jnp.stack([x[:, :DH], x[:, DH:]])

    w1s = jnp.stack([W1a, W1b])
    b1s = jnp.stack([b1a.reshape(1, D), b1b.reshape(1, D)])
    w2s = jnp.stack([W2a, W2b])
    b2s = jnp.stack([b2a.reshape(1, D), b2b.reshape(1, D)])

    def step(h, ws):
        w1, b1, w2, b2 = ws
        agg = _sc_gather_segsum(h, src_r, dst_r)
        return _tc_mlp(agg, h, w1, b1, w2, b2), None

    h2, _ = lax.scan(step, xh, (w1s, b1s, w2s, b2s))
    return _tc_pool(h2, batch_r)
